# Initial kernel scaffold; baseline (speedup 1.0000x reference)
import jax
import jax.numpy as jnp
from jax.experimental import pallas as pl


def kernel(edge_index, input, W_in0, W_rec0, W_in1, W_rec1):
    raise NotImplementedError("write your pallas kernel here")


# speedup vs baseline: 3.5678x; 3.5678x over previous
import functools

import jax
import jax.numpy as jnp
from jax import lax
from jax.experimental import pallas as pl
from jax.experimental.pallas import tpu as pltpu
from jax.experimental.pallas import tpu_sc as plsc

N = 10000
E = 320000
T = 8
F = 128
H = 128
LEAKAGE = 0.9

NC = 2
NS = 16
CHUNK = 128
CH_PER_SUB = 157
E_PER_SUB = CH_PER_SUB * CHUNK
EPAD = E_PER_SUB * NS
NP = 10240
ROWS_PER_SUB = NP // NS


def _sc_body(s0_hbm, s1_hbm, src_hbm, dst_hbm, a0_hbm, a1_hbm,
             sidx, didx, rows, acc, sem):
    c = lax.axis_index("c")
    s = lax.axis_index("s")

    def segsum(state_ref, out_ref):
        zv = jnp.zeros((16,), jnp.float32)

        def zrow(i, _):
            def zcol(j, _):
                rows[i, pl.ds(j * 16, 16)] = zv
                return 0
            return lax.fori_loop(0, H // 16, zcol, 0)

        lax.fori_loop(0, CHUNK, zrow, 0)
        zbase = s * ROWS_PER_SUB
        for k in range(ROWS_PER_SUB // CHUNK):
            pltpu.sync_copy(rows, acc.at[pl.ds(zbase + k * CHUNK, CHUNK)])
        plsc.subcore_barrier()

        def chunk(k, _):
            base = pl.multiple_of(s * E_PER_SUB + k * CHUNK, 8)
            pltpu.sync_copy(src_hbm.at[pl.ds(base, CHUNK)], sidx)
            pltpu.sync_copy(dst_hbm.at[pl.ds(base, CHUNK)], didx)
            pltpu.async_copy(state_ref.at[sidx], rows, sem).wait()
            pltpu.sync_copy(rows, acc.at[didx], add=True)
            return 0

        lax.fori_loop(0, CH_PER_SUB, chunk, 0)
        plsc.subcore_barrier()

        for k in range(ROWS_PER_SUB // CHUNK):
            r0 = zbase + k * CHUNK
            pltpu.sync_copy(acc.at[pl.ds(r0, CHUNK)], rows)
            pltpu.sync_copy(rows, out_ref.at[pl.ds(r0, CHUNK)])

    @pl.when(c == 0)
    def _():
        segsum(s0_hbm, a0_hbm)

    @pl.when(c == 1)
    def _():
        segsum(s1_hbm, a1_hbm)


_sc_segsum = pl.kernel(
    _sc_body,
    out_type=(jax.ShapeDtypeStruct((NP, H), jnp.float32),
              jax.ShapeDtypeStruct((NP, H), jnp.float32)),
    mesh=plsc.VectorSubcoreMesh(core_axis_name="c", subcore_axis_name="s",
                                num_cores=NC, num_subcores=NS),
    scratch_types=[
        pltpu.VMEM((CHUNK,), jnp.int32),
        pltpu.VMEM((CHUNK,), jnp.int32),
        pltpu.VMEM((CHUNK, H), jnp.float32),
        pltpu.VMEM_SHARED((NP, H), jnp.float32),
        pltpu.SemaphoreType.DMA,
    ],
)


def _tc_body(x_ref, a0_ref, a1_ref, s0_ref, s1_ref,
             wi0_ref, wr0_ref, wi1_ref, wr1_ref, o0_ref, o1_ref):
    dn = (((1,), (1,)), ((), ()))
    z0 = (lax.dot_general(x_ref[...], wi0_ref[...], dn,
                          preferred_element_type=jnp.float32)
          + lax.dot_general(a0_ref[...], wr0_ref[...], dn,
                            preferred_element_type=jnp.float32))
    s0n = LEAKAGE * jnp.tanh(z0) + (1.0 - LEAKAGE) * s0_ref[...]
    z1 = (lax.dot_general(s0n, wi1_ref[...], dn,
                          preferred_element_type=jnp.float32)
          + lax.dot_general(a1_ref[...], wr1_ref[...], dn,
                            preferred_element_type=jnp.float32))
    o0_ref[...] = s0n
    o1_ref[...] = LEAKAGE * jnp.tanh(z1) + (1.0 - LEAKAGE) * s1_ref[...]


_BN = 1000


def _tc_update(x, a0, a1, s0, s1, wi0, wr0, wi1, wr1):
    row_spec = pl.BlockSpec((_BN, H), lambda i: (i, 0))
    w_spec = pl.BlockSpec((H, H), lambda i: (0, 0))
    return pl.pallas_call(
        _tc_body,
        grid=(N // _BN,),
        in_specs=[row_spec, row_spec, row_spec, row_spec, row_spec,
                  w_spec, w_spec, w_spec, w_spec],
        out_specs=[row_spec, row_spec],
        out_shape=(jax.ShapeDtypeStruct((N, H), jnp.float32),
                   jax.ShapeDtypeStruct((N, H), jnp.float32)),
    )(x, a0, a1, s0, s1, wi0, wr0, wi1, wr1)


def kernel(edge_index, input, W_in0, W_rec0, W_in1, W_rec1):
    src = edge_index[0].astype(jnp.int32)
    dst = edge_index[1].astype(jnp.int32)
    pad = EPAD - E
    pid = jnp.arange(pad, dtype=jnp.int32)
    src_p = jnp.concatenate([src, pid % N])
    dst_p = jnp.concatenate([dst, N + pid % (NP - N)])

    zs = jnp.zeros((N, H), jnp.float32)
    za = jnp.zeros((NP, H), jnp.float32)
    s0, s1 = _tc_update(input[0], za, za, zs, zs, W_in0, W_rec0, W_in1, W_rec1)
    for t in range(1, T):
        a0, a1 = _sc_segsum(s0, s1, src_p, dst_p)
        s0, s1 = _tc_update(input[t], a0, a1, s0, s1,
                            W_in0, W_rec0, W_in1, W_rec1)
    return s1


# speedup vs baseline: 7.5453x; 2.1148x over previous
import functools

import jax
import jax.numpy as jnp
from jax import lax
from jax.experimental import pallas as pl
from jax.experimental.pallas import tpu as pltpu
from jax.experimental.pallas import tpu_sc as plsc

N = 10000
E = 320000
T = 8
F = 128
H = 128
LEAKAGE = 0.9

NC = 2
NS = 16
CHUNK = 128
IB = 32
NIB = 5
CH_PER_SUB = IB * NIB
E_PER_SUB = CH_PER_SUB * CHUNK
EPAD = E_PER_SUB * NS
NP = 10240
ROWS_PER_SUB = NP // NS


def _sc_body(s0_hbm, s1_hbm, src_hbm, dst_hbm, a0_hbm, a1_hbm,
             sidx, didx, rows, acc, sems):
    c = lax.axis_index("c")
    s = lax.axis_index("s")

    def segsum(state_ref, out_ref):
        zv = jnp.zeros((16,), jnp.float32)

        def zrow(i, _):
            for j in range(H // 16):
                rows[0, i, pl.ds(j * 16, 16)] = zv
            return 0

        lax.fori_loop(0, CHUNK, zrow, 0)
        zbase = s * ROWS_PER_SUB
        for k in range(ROWS_PER_SUB // CHUNK):
            pltpu.sync_copy(rows.at[0], acc.at[pl.ds(zbase + k * CHUNK, CHUNK)])
        plsc.subcore_barrier()

        def block(ib, _):
            base = s * CH_PER_SUB + ib * IB
            pltpu.sync_copy(src_hbm.at[pl.ds(base, IB)], sidx)
            pltpu.sync_copy(dst_hbm.at[pl.ds(base, IB)], didx)
            for b in range(2):
                pltpu.async_copy(state_ref.at[sidx.at[b]], rows.at[b],
                                 sems[b])
            for b in range(IB):
                r = b % 2
                pltpu.make_async_copy(state_ref.at[sidx.at[b]], rows.at[r],
                                      sems[r]).wait()
                pltpu.sync_copy(rows.at[r], acc.at[didx.at[b]], add=True)
                if b + 2 < IB:
                    pltpu.async_copy(state_ref.at[sidx.at[b + 2]], rows.at[r],
                                     sems[r])
            return 0

        lax.fori_loop(0, NIB, block, 0)
        plsc.subcore_barrier()

        nd = ROWS_PER_SUB // CHUNK
        for k in range(nd):
            r0 = zbase + k * CHUNK
            b = k % 2
            if k >= 2:
                pltpu.make_async_copy(rows.at[b], out_ref.at[pl.ds(r0, CHUNK)],
                                      sems[b]).wait()
            pltpu.sync_copy(acc.at[pl.ds(r0, CHUNK)], rows.at[b])
            pltpu.async_copy(rows.at[b], out_ref.at[pl.ds(r0, CHUNK)],
                             sems[b])
        for k in (nd - 2, nd - 1):
            pltpu.make_async_copy(rows.at[k % 2],
                                  out_ref.at[pl.ds(zbase, CHUNK)],
                                  sems[k % 2]).wait()

    @pl.when(c == 0)
    def _():
        segsum(s0_hbm, a0_hbm)

    @pl.when(c == 1)
    def _():
        segsum(s1_hbm, a1_hbm)


_sc_segsum = pl.kernel(
    _sc_body,
    out_type=(jax.ShapeDtypeStruct((NP, H), jnp.float32),
              jax.ShapeDtypeStruct((NP, H), jnp.float32)),
    mesh=plsc.VectorSubcoreMesh(core_axis_name="c", subcore_axis_name="s",
                                num_cores=NC, num_subcores=NS),
    scratch_types=[
        pltpu.VMEM((IB, CHUNK), jnp.int32),
        pltpu.VMEM((IB, CHUNK), jnp.int32),
        pltpu.VMEM((2, CHUNK, H), jnp.float32),
        pltpu.VMEM_SHARED((NP, H), jnp.float32),
        [pltpu.SemaphoreType.DMA] * 2,
    ],
)


def _tc_body(x_ref, a0_ref, a1_ref, s0_ref, s1_ref,
             wi0_ref, wr0_ref, wi1_ref, wr1_ref, o0_ref, o1_ref):
    dn = (((1,), (1,)), ((), ()))
    z0 = (lax.dot_general(x_ref[...], wi0_ref[...], dn,
                          preferred_element_type=jnp.float32)
          + lax.dot_general(a0_ref[...], wr0_ref[...], dn,
                            preferred_element_type=jnp.float32))
    s0n = LEAKAGE * jnp.tanh(z0) + (1.0 - LEAKAGE) * s0_ref[...]
    z1 = (lax.dot_general(s0n, wi1_ref[...], dn,
                          preferred_element_type=jnp.float32)
          + lax.dot_general(a1_ref[...], wr1_ref[...], dn,
                            preferred_element_type=jnp.float32))
    o0_ref[...] = s0n
    o1_ref[...] = LEAKAGE * jnp.tanh(z1) + (1.0 - LEAKAGE) * s1_ref[...]


_BN = 1000


def _tc_update(x, a0, a1, s0, s1, wi0, wr0, wi1, wr1):
    row_spec = pl.BlockSpec((_BN, H), lambda i: (i, 0))
    w_spec = pl.BlockSpec((H, H), lambda i: (0, 0))
    return pl.pallas_call(
        _tc_body,
        grid=(N // _BN,),
        in_specs=[row_spec, row_spec, row_spec, row_spec, row_spec,
                  w_spec, w_spec, w_spec, w_spec],
        out_specs=[row_spec, row_spec],
        out_shape=(jax.ShapeDtypeStruct((N, H), jnp.float32),
                   jax.ShapeDtypeStruct((N, H), jnp.float32)),
    )(x, a0, a1, s0, s1, wi0, wr0, wi1, wr1)


def kernel(edge_index, input, W_in0, W_rec0, W_in1, W_rec1):
    src = edge_index[0].astype(jnp.int32)
    dst = edge_index[1].astype(jnp.int32)
    pad = EPAD - E
    pid = jnp.arange(pad, dtype=jnp.int32)
    src_p = jnp.concatenate([src, pid % N]).reshape(EPAD // CHUNK, CHUNK)
    dst_p = jnp.concatenate([dst, N + pid % (NP - N)]).reshape(
        EPAD // CHUNK, CHUNK)

    zs = jnp.zeros((N, H), jnp.float32)
    za = jnp.zeros((NP, H), jnp.float32)
    s0, s1 = _tc_update(input[0], za, za, zs, zs, W_in0, W_rec0, W_in1, W_rec1)
    for t in range(1, T):
        a0, a1 = _sc_segsum(s0, s1, src_p, dst_p)
        s0, s1 = _tc_update(input[t], a0, a1, s0, s1,
                            W_in0, W_rec0, W_in1, W_rec1)
    return s1


# speedup vs baseline: 8.1083x; 1.0746x over previous
import functools

import jax
import jax.numpy as jnp
from jax import lax
from jax.experimental import pallas as pl
from jax.experimental.pallas import tpu as pltpu
from jax.experimental.pallas import tpu_sc as plsc

N = 10000
E = 320000
T = 8
F = 128
H = 128
LEAKAGE = 0.9

NC = 2
NS = 16
CHUNK = 96
NBUF = 3
IB = 24
NIB = 9
CH_PER_SUB = IB * NIB
E_PER_SUB = CH_PER_SUB * CHUNK
EPAD = E_PER_SUB * NS
NP = 10240
ROWS_PER_SUB = NP // NS
_ZD_SIZES = [CHUNK] * (ROWS_PER_SUB // CHUNK)
if ROWS_PER_SUB % CHUNK:
    _ZD_SIZES.append(ROWS_PER_SUB % CHUNK)


def _sc_body(s0_hbm, s1_hbm, src_hbm, dst_hbm, a0_hbm, a1_hbm,
             sidx_a, didx_a, sidx_b, didx_b, rows, acc, gsems, ssems):
    c = lax.axis_index("c")
    s = lax.axis_index("s")

    def segsum(state_ref, out_ref):
        zv = jnp.zeros((16,), jnp.float32)

        def zrow(i, _):
            for j in range(H // 16):
                rows[0, i, pl.ds(j * 16, 16)] = zv
            return 0

        lax.fori_loop(0, CHUNK, zrow, 0)
        zbase = s * ROWS_PER_SUB
        zoff = 0
        for sz in _ZD_SIZES:
            pltpu.sync_copy(rows.at[0].at[pl.ds(0, sz)],
                            acc.at[pl.ds(zbase + zoff, sz)])
            zoff += sz
        plsc.subcore_barrier()

        def run_block(ib, cur_s, cur_d, nxt_s, nxt_d, first, last):
            for b in range(IB):
                r = b % NBUF
                pltpu.make_async_copy(state_ref.at[cur_s.at[b]], rows.at[r],
                                      gsems[r]).wait()
                pltpu.async_copy(rows.at[r], acc.at[cur_d.at[b]], ssems[r],
                                 add=True)
                t = b + 2
                if not (last and t >= IB):
                    if not (first and b == 0):
                        pltpu.make_async_copy(rows.at[t % NBUF],
                                              acc.at[pl.ds(0, CHUNK)],
                                              ssems[t % NBUF]).wait()
                    row = cur_s.at[t] if t < IB else nxt_s.at[t - IB]
                    pltpu.async_copy(state_ref.at[row], rows.at[t % NBUF],
                                     gsems[t % NBUF])
                if b == IB - 3 and not last:
                    nbase = s * CH_PER_SUB + (ib + 1) * IB
                    pltpu.sync_copy(src_hbm.at[pl.ds(nbase, IB)], nxt_s)
                    pltpu.sync_copy(dst_hbm.at[pl.ds(nbase, IB)], nxt_d)

        base0 = s * CH_PER_SUB
        pltpu.sync_copy(src_hbm.at[pl.ds(base0, IB)], sidx_a)
        pltpu.sync_copy(dst_hbm.at[pl.ds(base0, IB)], didx_a)
        for b in range(2):
            pltpu.async_copy(state_ref.at[sidx_a.at[b]], rows.at[b],
                             gsems[b])

        run_block(0, sidx_a, didx_a, sidx_b, didx_b, True, False)

        def pair(m, _):
            run_block(2 * m + 1, sidx_b, didx_b, sidx_a, didx_a, False, False)
            run_block(2 * m + 2, sidx_a, didx_a, sidx_b, didx_b, False, False)
            return 0

        lax.fori_loop(0, (NIB - 3) // 2, pair, 0)
        run_block(NIB - 2, sidx_b, didx_b, sidx_a, didx_a, False, False)
        run_block(NIB - 1, sidx_a, didx_a, sidx_b, didx_b, False, True)

        for r in range(NBUF):
            pltpu.make_async_copy(rows.at[r], acc.at[pl.ds(0, CHUNK)],
                                  ssems[r]).wait()
        plsc.subcore_barrier()

        nd = len(_ZD_SIZES)
        doff = 0
        for k, sz in enumerate(_ZD_SIZES):
            b = k % 2
            if k >= 2:
                psz = _ZD_SIZES[k - 2]
                pltpu.make_async_copy(rows.at[b].at[pl.ds(0, psz)],
                                      out_ref.at[pl.ds(zbase, psz)],
                                      gsems[b]).wait()
            pltpu.sync_copy(acc.at[pl.ds(zbase + doff, sz)],
                            rows.at[b].at[pl.ds(0, sz)])
            pltpu.async_copy(rows.at[b].at[pl.ds(0, sz)],
                             out_ref.at[pl.ds(zbase + doff, sz)], gsems[b])
            doff += sz
        for k in (nd - 2, nd - 1):
            sz = _ZD_SIZES[k]
            pltpu.make_async_copy(rows.at[k % 2].at[pl.ds(0, sz)],
                                  out_ref.at[pl.ds(zbase, sz)],
                                  gsems[k % 2]).wait()

    @pl.when(c == 0)
    def _():
        segsum(s0_hbm, a0_hbm)

    @pl.when(c == 1)
    def _():
        segsum(s1_hbm, a1_hbm)


_sc_segsum = pl.kernel(
    _sc_body,
    out_type=(jax.ShapeDtypeStruct((NP, H), jnp.float32),
              jax.ShapeDtypeStruct((NP, H), jnp.float32)),
    mesh=plsc.VectorSubcoreMesh(core_axis_name="c", subcore_axis_name="s",
                                num_cores=NC, num_subcores=NS),
    scratch_types=[
        pltpu.VMEM((IB, CHUNK), jnp.int32),
        pltpu.VMEM((IB, CHUNK), jnp.int32),
        pltpu.VMEM((IB, CHUNK), jnp.int32),
        pltpu.VMEM((IB, CHUNK), jnp.int32),
        pltpu.VMEM((NBUF, CHUNK, H), jnp.float32),
        pltpu.VMEM_SHARED((NP, H), jnp.float32),
        [pltpu.SemaphoreType.DMA] * NBUF,
        [pltpu.SemaphoreType.DMA] * NBUF,
    ],
)


def _tc_body(x_ref, a0_ref, a1_ref, s0_ref, s1_ref,
             wi0_ref, wr0_ref, wi1_ref, wr1_ref, o0_ref, o1_ref):
    dn = (((1,), (1,)), ((), ()))
    z0 = (lax.dot_general(x_ref[...], wi0_ref[...], dn,
                          preferred_element_type=jnp.float32)
          + lax.dot_general(a0_ref[...], wr0_ref[...], dn,
                            preferred_element_type=jnp.float32))
    s0n = LEAKAGE * jnp.tanh(z0) + (1.0 - LEAKAGE) * s0_ref[...]
    z1 = (lax.dot_general(s0n, wi1_ref[...], dn,
                          preferred_element_type=jnp.float32)
          + lax.dot_general(a1_ref[...], wr1_ref[...], dn,
                            preferred_element_type=jnp.float32))
    o0_ref[...] = s0n
    o1_ref[...] = LEAKAGE * jnp.tanh(z1) + (1.0 - LEAKAGE) * s1_ref[...]


_BN = 1000


def _tc_update(x, a0, a1, s0, s1, wi0, wr0, wi1, wr1):
    row_spec = pl.BlockSpec((_BN, H), lambda i: (i, 0))
    w_spec = pl.BlockSpec((H, H), lambda i: (0, 0))
    return pl.pallas_call(
        _tc_body,
        grid=(N // _BN,),
        in_specs=[row_spec, row_spec, row_spec, row_spec, row_spec,
                  w_spec, w_spec, w_spec, w_spec],
        out_specs=[row_spec, row_spec],
        out_shape=(jax.ShapeDtypeStruct((N, H), jnp.float32),
                   jax.ShapeDtypeStruct((N, H), jnp.float32)),
    )(x, a0, a1, s0, s1, wi0, wr0, wi1, wr1)


def kernel(edge_index, input, W_in0, W_rec0, W_in1, W_rec1):
    src = edge_index[0].astype(jnp.int32)
    dst = edge_index[1].astype(jnp.int32)
    pad = EPAD - E
    pid = jnp.arange(pad, dtype=jnp.int32)
    src_p = jnp.concatenate([src, pid % N]).reshape(EPAD // CHUNK, CHUNK)
    dst_p = jnp.concatenate([dst, N + pid % (NP - N)]).reshape(
        EPAD // CHUNK, CHUNK)

    zs = jnp.zeros((N, H), jnp.float32)
    za = jnp.zeros((NP, H), jnp.float32)
    s0, s1 = _tc_update(input[0], za, za, zs, zs, W_in0, W_rec0, W_in1, W_rec1)
    for t in range(1, T):
        a0, a1 = _sc_segsum(s0, s1, src_p, dst_p)
        s0, s1 = _tc_update(input[t], a0, a1, s0, s1,
                            W_in0, W_rec0, W_in1, W_rec1)
    return s1
